# baseline (speedup 1.0000x reference)
import jax
import jax.numpy as jnp
from jax import lax
from jax.experimental import pallas as pl
from jax.experimental.pallas import tpu as pltpu

_N = 1_000_000
_D = 64
_R = 10_000
_NCHUNK = _N // _R
_NBUF = 4
_AHEAD = 3


def _body(a_hbm, b_hbm, o_hbm, abuf, bbuf, obuf, sem_a, sem_b, sem_o):
    i = pl.program_id(0)
    slot = lax.rem(i, _NBUF)

    def issue(c):
        s = lax.rem(c, _NBUF)
        pltpu.make_async_copy(
            a_hbm.at[pl.ds(c * _R, _R), :], abuf.at[s], sem_a.at[s]).start()
        pltpu.make_async_copy(
            b_hbm.at[pl.ds(c * _R, _R), :], bbuf.at[s], sem_b.at[s]).start()

    @pl.when(i == 0)
    def _():
        for k in range(_AHEAD):
            issue(k)

    @pl.when(i + _AHEAD < _NCHUNK)
    def _():
        issue(i + _AHEAD)

    pltpu.make_async_copy(
        a_hbm.at[pl.ds(i * _R, _R), :], abuf.at[slot], sem_a.at[slot]).wait()
    pltpu.make_async_copy(
        b_hbm.at[pl.ds(i * _R, _R), :], bbuf.at[slot], sem_b.at[slot]).wait()

    @pl.when(i >= _NBUF)
    def _():
        c = i - _NBUF
        pltpu.make_async_copy(
            obuf.at[slot], o_hbm.at[c], sem_o.at[slot]).wait()

    p = abuf[slot] * bbuf[slot]
    pt = p.T
    obuf[slot, 0, :] = jnp.sum(pt, axis=0)

    pltpu.make_async_copy(
        obuf.at[slot], o_hbm.at[i], sem_o.at[slot]).start()

    @pl.when(i == _NCHUNK - 1)
    def _():
        for k in range(_NBUF):
            c = _NCHUNK - _NBUF + k
            s = c % _NBUF
            pltpu.make_async_copy(
                obuf.at[s], o_hbm.at[c], sem_o.at[s]).wait()


def kernel(gut, git):
    out = pl.pallas_call(
        _body,
        grid=(_NCHUNK,),
        in_specs=[
            pl.BlockSpec(memory_space=pltpu.HBM),
            pl.BlockSpec(memory_space=pltpu.HBM),
        ],
        out_specs=pl.BlockSpec(memory_space=pltpu.HBM),
        out_shape=jax.ShapeDtypeStruct((_NCHUNK, 1, _R), jnp.float32),
        scratch_shapes=[
            pltpu.VMEM((_NBUF, _R, _D), jnp.float32),
            pltpu.VMEM((_NBUF, _R, _D), jnp.float32),
            pltpu.VMEM((_NBUF, 1, _R), jnp.float32),
            pltpu.SemaphoreType.DMA((_NBUF,)),
            pltpu.SemaphoreType.DMA((_NBUF,)),
            pltpu.SemaphoreType.DMA((_NBUF,)),
        ],
        compiler_params=pltpu.CompilerParams(
            dimension_semantics=("arbitrary",),
        ),
    )(gut, git)
    return out.reshape(_N)


# speedup vs baseline: 6.7474x; 6.7474x over previous
import jax
import jax.numpy as jnp
from jax.experimental import pallas as pl
from jax.experimental.pallas import tpu as pltpu

_N = 1_000_000
_D = 64
_BC = 32_768


def _body(a_ref, b_ref, o_ref):
    o_ref[...] = jnp.sum(a_ref[...] * b_ref[...], axis=0)


def kernel(gut, git):
    n_blocks = pl.cdiv(_N, _BC)
    out = pl.pallas_call(
        _body,
        grid=(n_blocks,),
        in_specs=[
            pl.BlockSpec((_D, _BC), lambda i: (0, i)),
            pl.BlockSpec((_D, _BC), lambda i: (0, i)),
        ],
        out_specs=pl.BlockSpec((_BC,), lambda i: (i,)),
        out_shape=jax.ShapeDtypeStruct((_N,), jnp.float32),
        compiler_params=pltpu.CompilerParams(
            dimension_semantics=("arbitrary",),
        ),
    )(gut.T, git.T)
    return out
